# baseline (speedup 1.0000x reference)
import functools

import jax
import jax.numpy as jnp
from jax import lax
from jax.experimental import pallas as pl
from jax.experimental.pallas import tpu as pltpu
from jax.experimental.pallas import tpu_sc as plsc

B = 16384
D = 32
NC, NS = 2, 16
NW = NC * NS
BPW = B // NW

_mesh = plsc.VectorSubcoreMesh(
    core_axis_name="c", subcore_axis_name="s", num_cores=NC, num_subcores=NS
)


@functools.partial(
    pl.kernel,
    out_type=jax.ShapeDtypeStruct((B, 7 * D), jnp.float32),
    mesh=_mesh,
    scratch_types=[
        pltpu.VMEM((BPW,), jnp.int32),
        pltpu.VMEM((BPW, D), jnp.float32),
        pltpu.SemaphoreType.DMA,
    ],
    compiler_params=pltpu.CompilerParams(use_tc_tiling_on_sc=False),
)
def _gather_concat(u, o, f0, f1, r, d, t, Wu, Wo, Wf, Wr, Wd, Wh,
                   out, idx_v, rows_v, sem):
    wid = lax.axis_index("s") * NC + lax.axis_index("c")
    base = wid * BPW
    pairs = ((u, Wu), (o, Wo), (f0, Wf), (f1, Wf), (r, Wr), (d, Wd), (t, Wh))
    for col, (idx_hbm, table) in enumerate(pairs):
        pltpu.sync_copy(idx_hbm.at[pl.ds(base, BPW)], idx_v)
        pltpu.async_copy(table.at[idx_v], rows_v, sem).wait()
        pltpu.sync_copy(rows_v, out.at[pl.ds(base, BPW), pl.ds(col * D, D)])


def kernel(user_id, organization, interested_fields_0, interested_fields_1,
           role, date, time, W_user, W_org, W_field, W_role, W_day, W_hour):
    idxs = [x.astype(jnp.int32) for x in
            (user_id, organization, interested_fields_0, interested_fields_1,
             role, date, time)]
    return _gather_concat(*idxs, W_user, W_org, W_field, W_role, W_day, W_hour)


# speedup vs baseline: 1.0174x; 1.0174x over previous
import functools

import jax
import jax.numpy as jnp
from jax import lax
from jax.experimental import pallas as pl
from jax.experimental.pallas import tpu as pltpu
from jax.experimental.pallas import tpu_sc as plsc

B = 16384
D = 32
NC, NS = 2, 16
NW = NC * NS
BPW = B // NW

_mesh = plsc.VectorSubcoreMesh(
    core_axis_name="c", subcore_axis_name="s", num_cores=NC, num_subcores=NS
)


@functools.partial(
    pl.kernel,
    out_type=jax.ShapeDtypeStruct((B, 7 * D), jnp.float32),
    mesh=_mesh,
    scratch_types=[
        pltpu.VMEM((7, BPW), jnp.int32),
        pltpu.VMEM((7, BPW, D), jnp.float32),
        pltpu.SemaphoreType.DMA,
        pltpu.SemaphoreType.DMA,
        pltpu.SemaphoreType.DMA,
    ],
    compiler_params=pltpu.CompilerParams(use_tc_tiling_on_sc=False),
)
def _gather_concat(u, o, f0, f1, r, d, t, Wu, Wo, Wf, Wr, Wd, Wh,
                   out, idx_v, rows_v, sem_i, sem_g, sem_o):
    wid = lax.axis_index("s") * NC + lax.axis_index("c")
    base = wid * BPW
    idx_hbm = (u, o, f0, f1, r, d, t)
    tables = (Wu, Wo, Wf, Wf, Wr, Wd, Wh)
    icps = [
        pltpu.async_copy(idx_hbm[i].at[pl.ds(base, BPW)], idx_v.at[i], sem_i)
        for i in range(7)
    ]
    for c in icps:
        c.wait()
    gcps = [
        pltpu.async_copy(tables[i].at[idx_v.at[i]], rows_v.at[i], sem_g)
        for i in range(7)
    ]
    for c in gcps:
        c.wait()
    ocps = [
        pltpu.async_copy(rows_v.at[i],
                         out.at[pl.ds(base, BPW), pl.ds(i * D, D)], sem_o)
        for i in range(7)
    ]
    for c in ocps:
        c.wait()


def kernel(user_id, organization, interested_fields_0, interested_fields_1,
           role, date, time, W_user, W_org, W_field, W_role, W_day, W_hour):
    idxs = [x.astype(jnp.int32) for x in
            (user_id, organization, interested_fields_0, interested_fields_1,
             role, date, time)]
    return _gather_concat(*idxs, W_user, W_org, W_field, W_role, W_day, W_hour)
